# baseline (speedup 1.0000x reference)
import dataclasses
import functools

import jax
import jax.numpy as jnp
from jax import lax
from jax.experimental import pallas as pl
from jax.experimental.pallas import tpu as pltpu
from jax.experimental.pallas import tpu_sc as plsc

N = 10000
E = 320000
D = 128
NP = 10240
NC = 2
NS = 16
NW = NC * NS
CHUNK = 128
CHUNKS = 80
W = 16
NWIN = CHUNKS // W
E_PAD = NW * CHUNKS * CHUNK
ROWS_PER_SUB = NP // NS

_mesh = plsc.VectorSubcoreMesh(core_axis_name="c", subcore_axis_name="s")

_cp = pltpu.CompilerParams()
if "needs_layout_passes" in pltpu.CompilerParams.__dataclass_fields__:
    _cp = dataclasses.replace(_cp, needs_layout_passes=False)


@functools.partial(
    pl.kernel,
    out_type=jax.ShapeDtypeStruct((NW, NP), jnp.float32),
    mesh=_mesh,
    scratch_types=[
        pltpu.VMEM((CHUNKS * CHUNK,), jnp.int32),
        pltpu.VMEM((NP,), jnp.float32),
    ],
    compiler_params=_cp,
)
def _sc_degree_hist(dst_hbm, out_hbm, dstv, hist):
    wid = lax.axis_index("s") * NC + lax.axis_index("c")
    pltpu.sync_copy(dst_hbm.at[wid], dstv)

    @pl.loop(0, NP, step=16)
    def _(i):
        hist[pl.ds(i, 16)] = jnp.zeros((16,), jnp.float32)

    ones = jnp.ones((16,), jnp.float32)

    @pl.loop(0, CHUNKS * CHUNK, step=16)
    def _(i):
        idx = dstv[pl.ds(i, 16)]
        plsc.addupdate_scatter(hist, [idx], ones)

    pltpu.sync_copy(hist, out_hbm.at[wid])


@functools.partial(
    pl.kernel,
    out_type=jax.ShapeDtypeStruct((NC, NP, D), jnp.float32),
    mesh=_mesh,
    scratch_types=[
        pltpu.VMEM((CHUNKS, CHUNK), jnp.int32),
        pltpu.VMEM((2, W, CHUNK), jnp.int32),
        pltpu.VMEM((CHUNK, D), jnp.float32),
        pltpu.VMEM((CHUNK, D), jnp.float32),
        pltpu.VMEM_SHARED((NP, D), jnp.float32),
        pltpu.SemaphoreType.DMA,
        pltpu.SemaphoreType.DMA,
        pltpu.SemaphoreType.DMA,
    ],
)
def _sc_scatter_accum(
    g_hbm, srci_hbm, dsti_hbm, z_hbm, out_hbm,
    srcv, dstw, bufa, bufb, acc, sema, semb, semi,
):
    c = lax.axis_index("c")
    s = lax.axis_index("s")
    wid = s * NC + c
    pltpu.sync_copy(srci_hbm.at[wid], srcv)
    pltpu.sync_copy(dsti_hbm.at[wid, pl.ds(0, W)], dstw.at[0])
    pltpu.sync_copy(z_hbm, acc.at[pl.ds(s * ROWS_PER_SUB, ROWS_PER_SUB)])
    plsc.subcore_barrier()

    pltpu.async_copy(g_hbm.at[srcv.at[0]], bufa, sema)

    @pl.loop(0, NWIN)
    def _(wi):
        base = wi * W
        pb = lax.rem(wi, 2)

        @pl.when(wi + 1 < NWIN)
        def _():
            pltpu.async_copy(
                dsti_hbm.at[wid, pl.ds((wi + 1) * W, W)], dstw.at[1 - pb], semi
            )

        @pl.loop(0, W, step=2)
        def _(k):
            j = base + k
            pltpu.make_async_copy(g_hbm.at[srcv.at[j]], bufa, sema).wait()
            pltpu.async_copy(g_hbm.at[srcv.at[j + 1]], bufb, semb)
            pltpu.sync_copy(bufa, acc.at[dstw.at[pb, k]], add=True)
            pltpu.make_async_copy(g_hbm.at[srcv.at[j + 1]], bufb, semb).wait()

            @pl.when(j + 2 < CHUNKS)
            def _():
                pltpu.async_copy(g_hbm.at[srcv.at[j + 2]], bufa, sema)

            pltpu.sync_copy(bufb, acc.at[dstw.at[pb, k + 1]], add=True)

        @pl.when(wi + 1 < NWIN)
        def _():
            pltpu.make_async_copy(
                dsti_hbm.at[wid, pl.ds((wi + 1) * W, W)], dstw.at[1 - pb], semi
            ).wait()

    plsc.subcore_barrier()
    pltpu.sync_copy(
        acc.at[pl.ds(s * ROWS_PER_SUB, ROWS_PER_SUB)],
        out_hbm.at[c, pl.ds(s * ROWS_PER_SUB, ROWS_PER_SUB)],
    )


_BLK1 = 1280


def _tc_scale_body(degp_ref, h_ref, g_ref, dinv_ref):
    deg = jnp.sum(degp_ref[...], axis=0) + 1.0
    dinv = lax.rsqrt(deg)[:, None]
    dinv_ref[...] = dinv
    g_ref[...] = h_ref[...] * dinv


def _tc_scale(deg_part, h_pad):
    return pl.pallas_call(
        _tc_scale_body,
        grid=(NP // _BLK1,),
        in_specs=[
            pl.BlockSpec((NW, _BLK1), lambda i: (0, i)),
            pl.BlockSpec((_BLK1, D), lambda i: (i, 0)),
        ],
        out_specs=[
            pl.BlockSpec((_BLK1, D), lambda i: (i, 0)),
            pl.BlockSpec((_BLK1, 1), lambda i: (i, 0)),
        ],
        out_shape=[
            jax.ShapeDtypeStruct((NP, D), jnp.float32),
            jax.ShapeDtypeStruct((NP, 1), jnp.float32),
        ],
    )(deg_part, h_pad)


_BLK2 = 2000


def _tc_combine_body(s0_ref, s1_ref, g_ref, dinv_ref, w_ref, b_ref, out_ref):
    agg = s0_ref[...] + s1_ref[...] + g_ref[...]
    agg = agg * dinv_ref[...]
    out_ref[...] = (
        lax.dot_general(
            agg,
            w_ref[...],
            (((1,), (1,)), ((), ())),
            precision=lax.Precision.HIGHEST,
            preferred_element_type=jnp.float32,
        )
        + b_ref[...][None, :]
    )


def _tc_combine(s0, s1, g, dinv, w, b):
    return pl.pallas_call(
        _tc_combine_body,
        grid=(N // _BLK2,),
        in_specs=[
            pl.BlockSpec((_BLK2, D), lambda i: (i, 0)),
            pl.BlockSpec((_BLK2, D), lambda i: (i, 0)),
            pl.BlockSpec((_BLK2, D), lambda i: (i, 0)),
            pl.BlockSpec((_BLK2, 1), lambda i: (i, 0)),
            pl.BlockSpec((D, D), lambda i: (0, 0)),
            pl.BlockSpec((D,), lambda i: (0,)),
        ],
        out_specs=pl.BlockSpec((_BLK2, D), lambda i: (i, 0)),
        out_shape=jax.ShapeDtypeStruct((N, D), jnp.float32),
    )(s0, s1, g, dinv, w, b)


def kernel(H, edge_index, W, b):
    src = edge_index[0]
    dst = edge_index[1]
    pad = jnp.full((E_PAD - E,), N, jnp.int32)
    src_p = jnp.concatenate([src, pad]).reshape(NW, CHUNKS, CHUNK)
    dst_p = jnp.concatenate([dst, pad]).reshape(NW, CHUNKS, CHUNK)
    dst_flat = dst_p.reshape(NW, CHUNKS * CHUNK)
    h_pad = jnp.pad(H, ((0, NP - N), (0, 0)))

    zeros = jnp.zeros((ROWS_PER_SUB, D), jnp.float32)
    deg_part = _sc_degree_hist(dst_flat)
    g, dinv = _tc_scale(deg_part, h_pad)
    s_part = _sc_scatter_accum(g, src_p, dst_p, zeros)
    return _tc_combine(s_part[0], s_part[1], g, dinv, W, b)


# speedup vs baseline: 2.8026x; 2.8026x over previous
import dataclasses
import functools

import jax
import jax.numpy as jnp
from jax import lax
from jax.experimental import pallas as pl
from jax.experimental.pallas import tpu as pltpu
from jax.experimental.pallas import tpu_sc as plsc

N = 10000
E = 320000
D = 128
NP = 10240
NC = 2
NS = 16
NW = NC * NS
CHUNK = 128
CHUNKS = 80
W = 16
NWIN = CHUNKS // W
E_PAD = NW * CHUNKS * CHUNK
ROWS_PER_SUB = NP // NS

_mesh = plsc.VectorSubcoreMesh(core_axis_name="c", subcore_axis_name="s")

_cp = pltpu.CompilerParams()
if "needs_layout_passes" in pltpu.CompilerParams.__dataclass_fields__:
    _cp = dataclasses.replace(_cp, needs_layout_passes=False)


@functools.partial(
    pl.kernel,
    out_type=jax.ShapeDtypeStruct((NW, NP), jnp.float32),
    mesh=_mesh,
    scratch_types=[
        pltpu.VMEM((CHUNKS * CHUNK,), jnp.int32),
        pltpu.VMEM((NP,), jnp.float32),
    ],
    compiler_params=_cp,
)
def _sc_degree_hist(dst_hbm, out_hbm, dstv, hist):
    wid = lax.axis_index("s") * NC + lax.axis_index("c")
    pltpu.sync_copy(dst_hbm.at[wid], dstv)

    @pl.loop(0, NP, step=16)
    def _(i):
        hist[pl.ds(i, 16)] = jnp.zeros((16,), jnp.float32)

    ones = jnp.ones((16,), jnp.float32)

    @pl.loop(0, CHUNKS * CHUNK, step=16)
    def _(i):
        idx = dstv[pl.ds(i, 16)]
        plsc.addupdate_scatter(hist, [idx], ones)

    pltpu.sync_copy(hist, out_hbm.at[wid])


@functools.partial(
    pl.kernel,
    out_type=jax.ShapeDtypeStruct((NC, NP, D), jnp.float32),
    mesh=_mesh,
    scratch_types=[
        pltpu.VMEM((CHUNKS, CHUNK), jnp.int32),
        pltpu.VMEM((2, W, CHUNK), jnp.int32),
        pltpu.VMEM((CHUNK, D), jnp.float32),
        pltpu.VMEM((CHUNK, D), jnp.float32),
        pltpu.VMEM_SHARED((NP, D), jnp.float32),
        pltpu.SemaphoreType.DMA,
        pltpu.SemaphoreType.DMA,
        pltpu.SemaphoreType.DMA,
    ],
)
def _sc_scatter_accum(
    g_hbm, srci_hbm, dsti_hbm, z_hbm, out_hbm,
    srcv, dstw, bufa, bufb, acc, sema, semb, semi,
):
    c = lax.axis_index("c")
    s = lax.axis_index("s")
    wid = s * NC + c
    pltpu.sync_copy(srci_hbm.at[wid], srcv)
    pltpu.sync_copy(dsti_hbm.at[wid, pl.ds(0, W)], dstw.at[0])
    pltpu.sync_copy(z_hbm, acc.at[pl.ds(s * ROWS_PER_SUB, ROWS_PER_SUB)])
    plsc.subcore_barrier()

    pltpu.async_copy(g_hbm.at[srcv.at[0]], bufa, sema)

    @pl.loop(0, NWIN)
    def _(wi):
        base = wi * W
        pb = lax.rem(wi, 2)

        @pl.when(wi + 1 < NWIN)
        def _():
            pltpu.async_copy(
                dsti_hbm.at[wid, pl.ds((wi + 1) * W, W)], dstw.at[1 - pb], semi
            )

        @pl.loop(0, W, step=2)
        def _(k):
            j = base + k
            pltpu.make_async_copy(g_hbm.at[srcv.at[j]], bufa, sema).wait()
            pltpu.async_copy(g_hbm.at[srcv.at[j + 1]], bufb, semb)
            pltpu.sync_copy(bufa, acc.at[dstw.at[pb, k]], add=True)
            pltpu.make_async_copy(g_hbm.at[srcv.at[j + 1]], bufb, semb).wait()

            @pl.when(j + 2 < CHUNKS)
            def _():
                pltpu.async_copy(g_hbm.at[srcv.at[j + 2]], bufa, sema)

            pltpu.sync_copy(bufb, acc.at[dstw.at[pb, k + 1]], add=True)

        @pl.when(wi + 1 < NWIN)
        def _():
            pltpu.make_async_copy(
                dsti_hbm.at[wid, pl.ds((wi + 1) * W, W)], dstw.at[1 - pb], semi
            ).wait()

    plsc.subcore_barrier()
    pltpu.sync_copy(
        acc.at[pl.ds(s * ROWS_PER_SUB, ROWS_PER_SUB)],
        out_hbm.at[c, pl.ds(s * ROWS_PER_SUB, ROWS_PER_SUB)],
    )


_BLK1 = 1280


def _tc_scale_body(degp_ref, h_ref, g_ref, dinv_ref):
    deg = jnp.sum(degp_ref[...], axis=0) + 1.0
    dinv = lax.rsqrt(deg)[:, None]
    dinv_ref[...] = dinv
    g_ref[...] = h_ref[...] * dinv


def _tc_scale(deg_part, h_pad):
    return pl.pallas_call(
        _tc_scale_body,
        grid=(NP // _BLK1,),
        in_specs=[
            pl.BlockSpec((NW, _BLK1), lambda i: (0, i)),
            pl.BlockSpec((_BLK1, D), lambda i: (i, 0)),
        ],
        out_specs=[
            pl.BlockSpec((_BLK1, D), lambda i: (i, 0)),
            pl.BlockSpec((_BLK1, 1), lambda i: (i, 0)),
        ],
        out_shape=[
            jax.ShapeDtypeStruct((NP, D), jnp.float32),
            jax.ShapeDtypeStruct((NP, 1), jnp.float32),
        ],
    )(deg_part, h_pad)


_BLK2 = 2000


def _tc_combine_body(s0_ref, s1_ref, g_ref, dinv_ref, w_ref, b_ref, out_ref):
    agg = s0_ref[...] + s1_ref[...] + g_ref[...]
    agg = agg * dinv_ref[...]
    out_ref[...] = (
        lax.dot_general(
            agg,
            w_ref[...],
            (((1,), (1,)), ((), ())),
            precision=lax.Precision.HIGHEST,
            preferred_element_type=jnp.float32,
        )
        + b_ref[...][None, :]
    )


def _tc_combine(s0, s1, g, dinv, w, b):
    return pl.pallas_call(
        _tc_combine_body,
        grid=(N // _BLK2,),
        in_specs=[
            pl.BlockSpec((_BLK2, D), lambda i: (i, 0)),
            pl.BlockSpec((_BLK2, D), lambda i: (i, 0)),
            pl.BlockSpec((_BLK2, D), lambda i: (i, 0)),
            pl.BlockSpec((_BLK2, 1), lambda i: (i, 0)),
            pl.BlockSpec((D, D), lambda i: (0, 0)),
            pl.BlockSpec((D,), lambda i: (0,)),
        ],
        out_specs=pl.BlockSpec((_BLK2, D), lambda i: (i, 0)),
        out_shape=jax.ShapeDtypeStruct((N, D), jnp.float32),
    )(s0, s1, g, dinv, w, b)


def kernel(H, edge_index, W, b):
    src = edge_index[0]
    dst = edge_index[1]
    pad = N + jnp.arange(E_PAD - E, dtype=jnp.int32) % (NP - N)
    src_p = jnp.concatenate([src, pad]).reshape(NW, CHUNKS, CHUNK)
    dst_p = jnp.concatenate([dst, pad]).reshape(NW, CHUNKS, CHUNK)
    dst_flat = dst_p.reshape(NW, CHUNKS * CHUNK)
    h_pad = jnp.pad(H, ((0, NP - N), (0, 0)))

    zeros = jnp.zeros((ROWS_PER_SUB, D), jnp.float32)
    deg_part = _sc_degree_hist(dst_flat)
    g, dinv = _tc_scale(deg_part, h_pad)
    s_part = _sc_scatter_accum(g, src_p, dst_p, zeros)
    return _tc_combine(s_part[0], s_part[1], g, dinv, W, b)


# speedup vs baseline: 2.9506x; 1.0528x over previous
import dataclasses
import functools

import jax
import jax.numpy as jnp
from jax import lax
from jax.experimental import pallas as pl
from jax.experimental.pallas import tpu as pltpu
from jax.experimental.pallas import tpu_sc as plsc

N = 10000
E = 320000
D = 128
NC = 2
NS = 16
NW = NC * NS
CHUNK = 125
CHUNKS = 80
E_SUB = CHUNKS * CHUNK
WIN = 16
NWIN = CHUNKS // WIN
NACC = 10112
ROWS_PER_SUB = NACC // NS

_mesh = plsc.VectorSubcoreMesh(core_axis_name="c", subcore_axis_name="s")

_cp = pltpu.CompilerParams()
if "needs_layout_passes" in pltpu.CompilerParams.__dataclass_fields__:
    _cp = dataclasses.replace(_cp, needs_layout_passes=False)


@functools.partial(
    pl.kernel,
    out_type=jax.ShapeDtypeStruct((NW, N), jnp.float32),
    mesh=_mesh,
    scratch_types=[
        pltpu.VMEM((E_SUB,), jnp.int32),
        pltpu.VMEM((N,), jnp.float32),
    ],
    compiler_params=_cp,
)
def _sc_degree_hist(dst_hbm, out_hbm, dstv, hist):
    wid = lax.axis_index("s") * NC + lax.axis_index("c")
    pltpu.sync_copy(dst_hbm.at[wid], dstv)

    @pl.loop(0, N, step=16)
    def _(i):
        hist[pl.ds(i, 16)] = jnp.zeros((16,), jnp.float32)

    ones = jnp.ones((16,), jnp.float32)

    @pl.loop(0, E_SUB, step=16)
    def _(i):
        idx = dstv[pl.ds(i, 16)]
        plsc.addupdate_scatter(hist, [idx], ones)

    pltpu.sync_copy(hist, out_hbm.at[wid])


@functools.partial(
    pl.kernel,
    out_type=jax.ShapeDtypeStruct((NC, NACC, D), jnp.float32),
    mesh=_mesh,
    scratch_types=[
        pltpu.VMEM((CHUNKS, CHUNK), jnp.int32),
        pltpu.VMEM((2, WIN, CHUNK), jnp.int32),
        pltpu.VMEM((CHUNK, D), jnp.float32),
        pltpu.VMEM((CHUNK, D), jnp.float32),
        pltpu.VMEM_SHARED((NACC, D), jnp.float32),
        pltpu.SemaphoreType.DMA,
        pltpu.SemaphoreType.DMA,
        pltpu.SemaphoreType.DMA,
    ],
)
def _sc_scatter_accum(
    ei_hbm, g_hbm, z_hbm, out_hbm,
    srcv, dstw, bufa, bufb, acc, sema, semb, semi,
):
    c = lax.axis_index("c")
    s = lax.axis_index("s")
    wid = s * NC + c
    pltpu.sync_copy(ei_hbm.at[0, wid], srcv)
    pltpu.sync_copy(ei_hbm.at[1, wid, pl.ds(0, WIN)], dstw.at[0])
    pltpu.sync_copy(z_hbm, acc.at[pl.ds(s * ROWS_PER_SUB, ROWS_PER_SUB)])
    plsc.subcore_barrier()

    pltpu.async_copy(g_hbm.at[srcv.at[0]], bufa, sema)

    @pl.loop(0, NWIN)
    def _(wi):
        base = wi * WIN
        pb = lax.rem(wi, 2)

        @pl.when(wi + 1 < NWIN)
        def _():
            pltpu.async_copy(
                ei_hbm.at[1, wid, pl.ds((wi + 1) * WIN, WIN)], dstw.at[1 - pb], semi
            )

        @pl.loop(0, WIN, step=2)
        def _(k):
            j = base + k
            pltpu.make_async_copy(g_hbm.at[srcv.at[j]], bufa, sema).wait()
            pltpu.async_copy(g_hbm.at[srcv.at[j + 1]], bufb, semb)
            pltpu.sync_copy(bufa, acc.at[dstw.at[pb, k]], add=True)
            pltpu.make_async_copy(g_hbm.at[srcv.at[j + 1]], bufb, semb).wait()

            @pl.when(j + 2 < CHUNKS)
            def _():
                pltpu.async_copy(g_hbm.at[srcv.at[j + 2]], bufa, sema)

            pltpu.sync_copy(bufb, acc.at[dstw.at[pb, k + 1]], add=True)

        @pl.when(wi + 1 < NWIN)
        def _():
            pltpu.make_async_copy(
                ei_hbm.at[1, wid, pl.ds((wi + 1) * WIN, WIN)], dstw.at[1 - pb], semi
            ).wait()

    plsc.subcore_barrier()
    pltpu.sync_copy(
        acc.at[pl.ds(s * ROWS_PER_SUB, ROWS_PER_SUB)],
        out_hbm.at[c, pl.ds(s * ROWS_PER_SUB, ROWS_PER_SUB)],
    )


def _tc_scale_body(degp_ref, h_ref, g_ref, dinv_ref):
    deg = jnp.sum(degp_ref[...], axis=0) + 1.0
    dinv = lax.rsqrt(deg)[:, None]
    dinv_ref[...] = dinv
    g_ref[...] = h_ref[...] * dinv


def _tc_scale(deg_part, h):
    return pl.pallas_call(
        _tc_scale_body,
        out_shape=[
            jax.ShapeDtypeStruct((N, D), jnp.float32),
            jax.ShapeDtypeStruct((N, 1), jnp.float32),
        ],
    )(deg_part, h)


def _tc_combine_body(sp_ref, g_ref, dinv_ref, w_ref, b_ref, out_ref):
    agg = sp_ref[0, :N] + sp_ref[1, :N] + g_ref[...]
    agg = agg * dinv_ref[...]
    out_ref[...] = (
        lax.dot_general(
            agg,
            w_ref[...],
            (((1,), (1,)), ((), ())),
            precision=lax.Precision.HIGHEST,
            preferred_element_type=jnp.float32,
        )
        + b_ref[...][None, :]
    )


def _tc_combine(s_part, g, dinv, w, b):
    return pl.pallas_call(
        _tc_combine_body,
        out_shape=jax.ShapeDtypeStruct((N, D), jnp.float32),
    )(s_part, g, dinv, w, b)


def kernel(H, edge_index, W, b):
    ei = edge_index.reshape(2, NW, CHUNKS, CHUNK)
    dst_flat = edge_index[1].reshape(NW, E_SUB)
    zeros = jnp.zeros((ROWS_PER_SUB, D), jnp.float32)

    deg_part = _sc_degree_hist(dst_flat)
    g, dinv = _tc_scale(deg_part, H)
    s_part = _sc_scatter_accum(ei, g, zeros)
    return _tc_combine(s_part, g, dinv, W, b)
